# baseline (speedup 1.0000x reference)
import functools

import jax
import jax.numpy as jnp
from jax import lax
from jax.experimental import pallas as pl
from jax.experimental.pallas import tpu as pltpu
from jax.experimental.pallas import tpu_sc as plsc

_NUM_CORES = 2
_NUM_SUBCORES = 16
_NW = _NUM_CORES * _NUM_SUBCORES
_LANES = 16

_G = 4
_NIDX = _G * 128
_PAD = 8
_NBUF = 3


def _body(x_hbm, pe_hbm, out_hbm, x_v, idx_v, rows_v,
          xsem, gsem, wsem, *, idx_per_worker):
    wid = lax.axis_index("c") * _NUM_SUBCORES + lax.axis_index("s")
    nchunk = idx_per_worker // _NIDX
    base0 = wid * idx_per_worker
    lane = lax.iota(jnp.int32, _LANES)
    is_odd = (lane & 1) == 1

    def fire_xload(cc, b):
        pltpu.async_copy(x_hbm.at[pl.ds(base0 + cc * _NIDX, _NIDX)],
                         x_v.at[b].at[pl.ds(_PAD, _NIDX)], xsem.at[b])

    def wait_xload(b):
        pltpu.make_async_copy(x_hbm.at[pl.ds(0, _NIDX)],
                              x_v.at[b].at[pl.ds(_PAD, _NIDX)],
                              xsem.at[b]).wait()

    def compute_idx(b):
        for i in range(_NIDX // _LANES):
            v = x_v[b, pl.ds(_PAD + i * _LANES, _LANES)]
            vm1 = x_v[b, pl.ds(_PAD - 1 + i * _LANES, _LANES)]
            fused = jnp.where(vm1 == 0, 0, v - vm1 + 1)
            idx_v[b, i // 8, pl.ds((i % 8) * _LANES, _LANES)] = (
                jnp.where(is_odd, fused, v))

    def fire_gathers(b):
        for j in range(_G):
            pltpu.async_copy(pe_hbm.at[idx_v.at[b].at[j]],
                             rows_v.at[b].at[pl.ds(j * 128, 128)],
                             gsem.at[b])

    def drain_gathers(b):
        pltpu.make_async_copy(pe_hbm.at[pl.ds(0, _NIDX)], rows_v.at[b],
                              gsem.at[b]).wait()

    def fire_writeback(cc, b):
        pltpu.async_copy(rows_v.at[b],
                         out_hbm.at[pl.ds(base0 + cc * _NIDX, _NIDX)],
                         wsem.at[b])

    def drain_writeback(b):
        pltpu.make_async_copy(rows_v.at[b], out_hbm.at[pl.ds(0, _NIDX)],
                              wsem.at[b]).wait()

    fire_xload(0, 0)

    def loop_body(c3, _):
        for b in range(_NBUF):
            cc = c3 * _NBUF + b
            p2 = (b - 2) % _NBUF
            wait_xload(b)
            compute_idx(b)
            if b == 2:
                drain_gathers(p2)
                fire_writeback(cc - 2, p2)
            else:
                @pl.when(c3 >= 1)
                def _():
                    drain_gathers(p2)
                    fire_writeback(cc - 2, p2)

            fire_xload(cc + 1, (b + 1) % _NBUF)

            @pl.when(c3 >= 1)
            def _():
                drain_writeback(b)

            fire_gathers(b)
        return ()

    nloop = (nchunk - 1) // _NBUF
    lax.fori_loop(0, nloop, loop_body, ())

    last = nchunk - 1
    wait_xload(0)
    compute_idx(0)
    drain_gathers(1)
    fire_writeback(last - 2, 1)
    drain_writeback(0)
    fire_gathers(0)
    drain_gathers(2)
    fire_writeback(last - 1, 2)
    drain_gathers(0)
    fire_writeback(last, 0)
    for b in range(_NBUF):
        drain_writeback(b)


def kernel(x, position_embedding):
    s0, s1, _ = x.shape
    n_idx = s0 * s1 * 2
    idx_per_worker = n_idx // _NW
    x_flat = x.astype(jnp.int32).reshape(-1)
    pe = position_embedding.astype(jnp.float32)

    mesh = plsc.VectorSubcoreMesh(
        core_axis_name="c", subcore_axis_name="s",
        num_cores=_NUM_CORES, num_subcores=_NUM_SUBCORES)
    k = pl.kernel(
        functools.partial(_body, idx_per_worker=idx_per_worker),
        out_type=jax.ShapeDtypeStruct((n_idx, 64), jnp.float32),
        mesh=mesh,
        compiler_params=pltpu.CompilerParams(use_tc_tiling_on_sc=False),
        scratch_types=[
            pltpu.VMEM((_NBUF, _PAD + _NIDX), jnp.int32),
            pltpu.VMEM((_NBUF, _G, 128), jnp.int32),
            pltpu.VMEM((_NBUF, _NIDX, 64), jnp.float32),
            pltpu.SemaphoreType.DMA((_NBUF,)),
            pltpu.SemaphoreType.DMA((_NBUF,)),
            pltpu.SemaphoreType.DMA((_NBUF,)),
        ],
    )
    out = k(x_flat, pe)
    return out.reshape(s0, s1, 128)


# speedup vs baseline: 1.0008x; 1.0008x over previous
import functools

import jax
import jax.numpy as jnp
from jax import lax
from jax.experimental import pallas as pl
from jax.experimental.pallas import tpu as pltpu
from jax.experimental.pallas import tpu_sc as plsc

_NUM_CORES = 2
_NUM_SUBCORES = 16
_NW = _NUM_CORES * _NUM_SUBCORES
_LANES = 16

_G = 4
_NIDX = _G * 128
_PAD = 8
_NBUF = 3


def _body(x_hbm, pe_hbm, out_hbm, x_v, idx_v, rows_v,
          xsem, gsem, wsem, *, idx_per_worker):
    wid = lax.axis_index("c") * _NUM_SUBCORES + lax.axis_index("s")
    nchunk = idx_per_worker // _NIDX
    base0 = wid * idx_per_worker
    lane = lax.iota(jnp.int32, _LANES)
    is_odd = (lane & 1) == 1

    row0 = base0 // 128

    def fire_xload(cc, b):
        for j in range(_G):
            pltpu.async_copy(x_hbm.at[row0 + cc * _G + j],
                             x_v.at[b].at[pl.ds(_PAD + j * 128, 128)],
                             xsem.at[b])

    def wait_xload(b):
        for j in range(_G):
            pltpu.make_async_copy(x_hbm.at[0],
                                  x_v.at[b].at[pl.ds(_PAD + j * 128, 128)],
                                  xsem.at[b]).wait()

    def compute_idx(b):
        for i in range(_NIDX // _LANES):
            v = x_v[b, pl.ds(_PAD + i * _LANES, _LANES)]
            vm1 = x_v[b, pl.ds(_PAD - 1 + i * _LANES, _LANES)]
            fused = jnp.where(vm1 == 0, 0, v - vm1 + 1)
            idx_v[b, i // 8, pl.ds((i % 8) * _LANES, _LANES)] = (
                jnp.where(is_odd, fused, v))

    def fire_gathers(b):
        for j in range(_G):
            pltpu.async_copy(pe_hbm.at[idx_v.at[b].at[j]],
                             rows_v.at[b].at[pl.ds(j * 128, 128)],
                             gsem.at[b])

    def drain_gathers(b):
        pltpu.make_async_copy(pe_hbm.at[pl.ds(0, _NIDX)], rows_v.at[b],
                              gsem.at[b]).wait()

    def fire_writeback(cc, b):
        pltpu.async_copy(rows_v.at[b],
                         out_hbm.at[pl.ds(base0 + cc * _NIDX, _NIDX)],
                         wsem.at[b])

    def drain_writeback(b):
        pltpu.make_async_copy(rows_v.at[b], out_hbm.at[pl.ds(0, _NIDX)],
                              wsem.at[b]).wait()

    fire_xload(0, 0)

    def loop_body(c3, _):
        for b in range(_NBUF):
            cc = c3 * _NBUF + b
            p2 = (b - 2) % _NBUF
            wait_xload(b)
            compute_idx(b)
            if b == 2:
                drain_gathers(p2)
                fire_writeback(cc - 2, p2)
            else:
                @pl.when(c3 >= 1)
                def _():
                    drain_gathers(p2)
                    fire_writeback(cc - 2, p2)

            fire_xload(cc + 1, (b + 1) % _NBUF)

            @pl.when(c3 >= 1)
            def _():
                drain_writeback(b)

            fire_gathers(b)
        return ()

    nloop = (nchunk - 1) // _NBUF
    lax.fori_loop(0, nloop, loop_body, ())

    last = nchunk - 1
    wait_xload(0)
    compute_idx(0)
    drain_gathers(1)
    fire_writeback(last - 2, 1)
    drain_writeback(0)
    fire_gathers(0)
    drain_gathers(2)
    fire_writeback(last - 1, 2)
    drain_gathers(0)
    fire_writeback(last, 0)
    for b in range(_NBUF):
        drain_writeback(b)


def kernel(x, position_embedding):
    s0, s1, _ = x.shape
    n_idx = s0 * s1 * 2
    idx_per_worker = n_idx // _NW
    x_flat = x.astype(jnp.int32).reshape(-1, 128)
    pe = position_embedding.astype(jnp.float32)

    mesh = plsc.VectorSubcoreMesh(
        core_axis_name="c", subcore_axis_name="s",
        num_cores=_NUM_CORES, num_subcores=_NUM_SUBCORES)
    k = pl.kernel(
        functools.partial(_body, idx_per_worker=idx_per_worker),
        out_type=jax.ShapeDtypeStruct((n_idx, 64), jnp.float32),
        mesh=mesh,
        compiler_params=pltpu.CompilerParams(use_tc_tiling_on_sc=False),
        scratch_types=[
            pltpu.VMEM((_NBUF, _PAD + _NIDX), jnp.int32),
            pltpu.VMEM((_NBUF, _G, 128), jnp.int32),
            pltpu.VMEM((_NBUF, _NIDX, 64), jnp.float32),
            pltpu.SemaphoreType.DMA((_NBUF,)),
            pltpu.SemaphoreType.DMA((_NBUF,)),
            pltpu.SemaphoreType.DMA((_NBUF,)),
        ],
    )
    out = k(x_flat, pe)
    return out.reshape(s0, s1, 128)


# speedup vs baseline: 3.6886x; 3.6858x over previous
import functools

import jax
import jax.numpy as jnp
from jax import lax
from jax.experimental import pallas as pl
from jax.experimental.pallas import tpu as pltpu
from jax.experimental.pallas import tpu_sc as plsc

_NUM_CORES = 2
_NUM_SUBCORES = 16
_NW = _NUM_CORES * _NUM_SUBCORES
_LANES = 16

_G = 2
_R = _G * 128
_NBUF = 3


def _body(xab_hbm, pe_hbm, out_hbm, x_v, bi_v, fbuf, bbuf,
          xsem, gsem, wsem, *, rows_per_worker):
    wid = lax.axis_index("c") * _NUM_SUBCORES + lax.axis_index("s")
    nchunk = rows_per_worker // _R
    blk0 = wid * (rows_per_worker // 128)

    def fire_xload(cc, b):
        pltpu.async_copy(xab_hbm.at[pl.ds(blk0 + cc * _G, _G)], x_v.at[b],
                         xsem.at[b])

    def wait_xload(b):
        pltpu.make_async_copy(xab_hbm.at[pl.ds(0, _G)], x_v.at[b],
                              xsem.at[b]).wait()

    def compute_bwd(b):
        for j in range(_G):
            for k in range(128 // _LANES):
                sl = pl.ds(k * _LANES, _LANES)
                a = x_v[b, j, 0, sl]
                bb = x_v[b, j, 1, sl]
                bi_v[b, j, sl] = jnp.where(a == 0, 0, bb - a + 1)

    def fire_gathers(b):
        for j in range(_G):
            dst = pl.ds(j * 128, 128)
            pltpu.async_copy(pe_hbm.at[x_v.at[b].at[j].at[0]],
                             fbuf.at[b].at[dst], gsem.at[b])
            pltpu.async_copy(pe_hbm.at[bi_v.at[b].at[j]],
                             bbuf.at[b].at[dst], gsem.at[b])

    def drain_gathers(b):
        pltpu.make_async_copy(pe_hbm.at[pl.ds(0, _R)], fbuf.at[b],
                              gsem.at[b]).wait()
        pltpu.make_async_copy(pe_hbm.at[pl.ds(0, _R)], bbuf.at[b],
                              gsem.at[b]).wait()

    def fire_writeback(cc, b):
        base = (blk0 + cc * _G) * 128
        pltpu.async_copy(fbuf.at[b], out_hbm.at[pl.ds(base, _R), 0],
                         wsem.at[b, 0])
        pltpu.async_copy(bbuf.at[b], out_hbm.at[pl.ds(base, _R), 1],
                         wsem.at[b, 1])

    def drain_writeback(b):
        pltpu.make_async_copy(fbuf.at[b], out_hbm.at[pl.ds(0, _R), 0],
                              wsem.at[b, 0]).wait()
        pltpu.make_async_copy(bbuf.at[b], out_hbm.at[pl.ds(0, _R), 1],
                              wsem.at[b, 1]).wait()

    fire_xload(0, 0)

    def loop_body(c3, _):
        for b in range(_NBUF):
            cc = c3 * _NBUF + b
            p2 = (b - 2) % _NBUF
            wait_xload(b)
            compute_bwd(b)
            if b == 2:
                drain_gathers(p2)
                fire_writeback(cc - 2, p2)
            else:
                @pl.when(c3 >= 1)
                def _():
                    drain_gathers(p2)
                    fire_writeback(cc - 2, p2)

            fire_xload(cc + 1, (b + 1) % _NBUF)

            @pl.when(c3 >= 1)
            def _():
                drain_writeback(b)

            fire_gathers(b)
        return ()

    nloop = (nchunk - 1) // _NBUF
    lax.fori_loop(0, nloop, loop_body, ())

    last = nchunk - 1
    wait_xload(0)
    compute_bwd(0)
    drain_gathers(1)
    fire_writeback(last - 2, 1)
    drain_writeback(0)
    fire_gathers(0)
    drain_gathers(2)
    fire_writeback(last - 1, 2)
    drain_gathers(0)
    fire_writeback(last, 0)
    for b in range(_NBUF):
        drain_writeback(b)


def kernel(x, position_embedding):
    s0, s1, _ = x.shape
    b_total = s0 * s1
    rows_per_worker = b_total // _NW
    xi = x.astype(jnp.int32)
    xab = xi.reshape(-1, 128, 2).transpose(0, 2, 1)
    pe = position_embedding.astype(jnp.float32)

    mesh = plsc.VectorSubcoreMesh(
        core_axis_name="c", subcore_axis_name="s",
        num_cores=_NUM_CORES, num_subcores=_NUM_SUBCORES)
    k = pl.kernel(
        functools.partial(_body, rows_per_worker=rows_per_worker),
        out_type=jax.ShapeDtypeStruct((b_total, 2, 64), jnp.float32),
        mesh=mesh,
        compiler_params=pltpu.CompilerParams(use_tc_tiling_on_sc=False),
        scratch_types=[
            pltpu.VMEM((_NBUF, _G, 2, 128), jnp.int32),
            pltpu.VMEM((_NBUF, _G, 128), jnp.int32),
            pltpu.VMEM((_NBUF, _R, 64), jnp.float32),
            pltpu.VMEM((_NBUF, _R, 64), jnp.float32),
            pltpu.SemaphoreType.DMA((_NBUF,)),
            pltpu.SemaphoreType.DMA((_NBUF,)),
            pltpu.SemaphoreType.DMA((_NBUF, 2)),
        ],
    )
    out = k(xab, pe)
    return out.reshape(s0, s1, 128)


# speedup vs baseline: 3.7426x; 1.0146x over previous
import functools

import jax
import jax.numpy as jnp
from jax import lax
from jax.experimental import pallas as pl
from jax.experimental.pallas import tpu as pltpu
from jax.experimental.pallas import tpu_sc as plsc

_NUM_CORES = 2
_NUM_SUBCORES = 16
_NW = _NUM_CORES * _NUM_SUBCORES
_LANES = 16

_R = 128
_NBUF = 6


def _body(xab_hbm, pe_hbm, out_hbm, x_v, bi_v, fbuf, bbuf,
          xsem, gsem, wsem, *, rows_per_worker):
    wid = lax.axis_index("c") * _NUM_SUBCORES + lax.axis_index("s")
    nchunk = rows_per_worker // _R
    blk0 = wid * (rows_per_worker // 128)

    def fire_xload(cc, b):
        pltpu.async_copy(xab_hbm.at[blk0 + cc], x_v.at[b], xsem.at[b])

    def wait_xload(b):
        pltpu.make_async_copy(xab_hbm.at[0], x_v.at[b], xsem.at[b]).wait()

    def compute_bwd(b):
        for k in range(128 // _LANES):
            sl = pl.ds(k * _LANES, _LANES)
            a = x_v[b, 0, sl]
            bb = x_v[b, 1, sl]
            bi_v[b, sl] = jnp.where(a == 0, 0, bb - a + 1)

    def fire_gathers(b):
        pltpu.async_copy(pe_hbm.at[x_v.at[b].at[0]], fbuf.at[b], gsem.at[b])
        pltpu.async_copy(pe_hbm.at[bi_v.at[b]], bbuf.at[b], gsem.at[b])

    def drain_gathers(b):
        pltpu.make_async_copy(pe_hbm.at[pl.ds(0, _R)], fbuf.at[b],
                              gsem.at[b]).wait()
        pltpu.make_async_copy(pe_hbm.at[pl.ds(0, _R)], bbuf.at[b],
                              gsem.at[b]).wait()

    def fire_writeback(cc, b):
        base = (blk0 + cc) * 128
        pltpu.async_copy(fbuf.at[b], out_hbm.at[pl.ds(base, _R), 0],
                         wsem.at[b, 0])
        pltpu.async_copy(bbuf.at[b], out_hbm.at[pl.ds(base, _R), 1],
                         wsem.at[b, 1])

    def drain_writeback(b):
        pltpu.make_async_copy(fbuf.at[b], out_hbm.at[pl.ds(0, _R), 0],
                              wsem.at[b, 0]).wait()
        pltpu.make_async_copy(bbuf.at[b], out_hbm.at[pl.ds(0, _R), 1],
                              wsem.at[b, 1]).wait()

    fire_xload(0, 0)

    def loop_body(c6, _):
        for b in range(_NBUF):
            cc = c6 * _NBUF + b
            p5 = (b + 1) % _NBUF
            wait_xload(b)
            compute_bwd(b)
            if b == _NBUF - 1:
                drain_gathers(p5)
                fire_writeback(cc - 5, p5)
            else:
                @pl.when(c6 >= 1)
                def _():
                    drain_gathers(p5)
                    fire_writeback(cc - 5, p5)

            fire_xload(cc + 1, p5)

            @pl.when(c6 >= 1)
            def _():
                drain_writeback(b)

            fire_gathers(b)
        return ()

    nloop = (nchunk - 2) // _NBUF
    lax.fori_loop(0, nloop, loop_body, ())

    c_a = nchunk - 2
    wait_xload(0)
    compute_bwd(0)
    drain_gathers(1)
    fire_writeback(c_a - 5, 1)
    fire_xload(c_a + 1, 1)
    drain_writeback(0)
    fire_gathers(0)

    c_b = nchunk - 1
    wait_xload(1)
    compute_bwd(1)
    drain_gathers(2)
    fire_writeback(c_b - 5, 2)
    drain_writeback(1)
    fire_gathers(1)

    for cc, b in ((c_b - 4, 3), (c_b - 3, 4), (c_b - 2, 5),
                  (c_a, 0), (c_b, 1)):
        drain_gathers(b)
        fire_writeback(cc, b)
    for b in range(_NBUF):
        drain_writeback(b)


def kernel(x, position_embedding):
    s0, s1, _ = x.shape
    b_total = s0 * s1
    rows_per_worker = b_total // _NW
    xi = x.astype(jnp.int32)
    xab = xi.reshape(-1, 128, 2).transpose(0, 2, 1)
    pe = position_embedding.astype(jnp.float32)

    mesh = plsc.VectorSubcoreMesh(
        core_axis_name="c", subcore_axis_name="s",
        num_cores=_NUM_CORES, num_subcores=_NUM_SUBCORES)
    k = pl.kernel(
        functools.partial(_body, rows_per_worker=rows_per_worker),
        out_type=jax.ShapeDtypeStruct((b_total, 2, 64), jnp.float32),
        mesh=mesh,
        compiler_params=pltpu.CompilerParams(use_tc_tiling_on_sc=False),
        scratch_types=[
            pltpu.VMEM((_NBUF, 2, 128), jnp.int32),
            pltpu.VMEM((_NBUF, 128), jnp.int32),
            pltpu.VMEM((_NBUF, _R, 64), jnp.float32),
            pltpu.VMEM((_NBUF, _R, 64), jnp.float32),
            pltpu.SemaphoreType.DMA((_NBUF,)),
            pltpu.SemaphoreType.DMA((_NBUF,)),
            pltpu.SemaphoreType.DMA((_NBUF, 2)),
        ],
    )
    out = k(xab, pe)
    return out.reshape(s0, s1, 128)
